# Initial kernel scaffold; baseline (speedup 1.0000x reference)
import jax
import jax.numpy as jnp
from jax.experimental import pallas as pl


def kernel(sc_matrix, W1, b1, W2, b2, lnE_g, lnE_b, fc1_W, fc1_b, ln1_g, ln1_b, fc2_W, fc2_b, ln2_g, ln2_b, fc3_W, fc3_b):
    raise NotImplementedError("write your pallas kernel here")


# speedup vs baseline: 900.0952x; 900.0952x over previous
import jax
import jax.numpy as jnp
from jax import lax
from jax.experimental import pallas as pl

N = 400
B = 4
D = 128
EPS = 1e-5
_F32 = jnp.float32


def _ln(x, g, b):
    mu = jnp.mean(x, axis=-1, keepdims=True)
    var = jnp.mean((x - mu) ** 2, axis=-1, keepdims=True)
    return (x - mu) * lax.rsqrt(var + EPS) * g + b


def _dot(a, c):
    return jnp.dot(a, c, preferred_element_type=_F32,
                   precision=lax.Precision.HIGHEST)


def _tdot(a, c):
    return lax.dot_general(a, c, (((0,), (0,)), ((), ())),
                           preferred_element_type=_F32,
                           precision=lax.Precision.HIGHEST)


def _fused_kernel(sc_ref, W1_ref, b1_ref, W2_ref, b2_ref, lnEg_ref, lnEb_ref,
                  fc1W_ref, fc1b_ref, ln1g_ref, ln1b_ref,
                  fc2W_ref, fc2b_ref, ln2g_ref, ln2b_ref,
                  fc3W_ref, fc3b_ref,
                  logits_ref, zp_ref):
    b = pl.program_id(0)
    S = sc_ref[0]

    deg = _tdot(S, jnp.ones((N, 1), _F32))
    dinv = jnp.where(deg > 0, lax.rsqrt(deg), 0.0)

    h = _dot(S, W1_ref[...])
    x = jnp.maximum(_tdot(S, h * dinv) * dinv + b1_ref[...], 0.0)
    h = _dot(x, W2_ref[...])
    x = _tdot(S, h * dinv) * dinv + b2_ref[...]
    x = _ln(x, lnEg_ref[...], lnEb_ref[...])
    zp_ref[pl.ds(b, 1), :] = jnp.mean(x, axis=0, keepdims=True)

    @pl.when(b == B - 1)
    def _head():
        z = zp_ref[...]
        hh = _dot(z, fc1W_ref[...]) + fc1b_ref[...]
        hh = jnp.maximum(_ln(hh, ln1g_ref[...], ln1b_ref[...]), 0.0)
        hh = _dot(hh, fc2W_ref[...]) + fc2b_ref[...]
        hh = jnp.maximum(_ln(hh, ln2g_ref[...], ln2b_ref[...]), 0.0)
        logits_ref[...] = _dot(hh, fc3W_ref[...]) + fc3b_ref[...]


def _full(shape):
    return pl.BlockSpec(shape, lambda b: (0,) * len(shape))


def kernel(sc_matrix, W1, b1, W2, b2, lnE_g, lnE_b, fc1_W, fc1_b, ln1_g,
           ln1_b, fc2_W, fc2_b, ln2_g, ln2_b, fc3_W, fc3_b):
    r2 = lambda v: v.reshape(1, -1)
    logits, zp = pl.pallas_call(
        _fused_kernel,
        grid=(B,),
        in_specs=[
            pl.BlockSpec((1, N, N), lambda b: (b, 0, 0)),
            _full((N, D)), _full((1, D)), _full((D, D)), _full((1, D)),
            _full((1, D)), _full((1, D)),
            _full((D, 128)), _full((1, 128)), _full((1, 128)), _full((1, 128)),
            _full((128, 64)), _full((1, 64)), _full((1, 64)), _full((1, 64)),
            _full((64, 4)), _full((1, 4)),
        ],
        out_specs=[
            pl.BlockSpec((B, 4), lambda b: (0, 0)),
            pl.BlockSpec((B, D), lambda b: (0, 0)),
        ],
        out_shape=[
            jax.ShapeDtypeStruct((B, 4), _F32),
            jax.ShapeDtypeStruct((B, D), _F32),
        ],
    )(sc_matrix, W1, r2(b1), W2, r2(b2), r2(lnE_g), r2(lnE_b),
      fc1_W, r2(fc1_b), r2(ln1_g), r2(ln1_b),
      fc2_W, r2(fc2_b), r2(ln2_g), r2(ln2_b),
      fc3_W, r2(fc3_b))
    return (logits, zp)


# speedup vs baseline: 2006.1921x; 2.2289x over previous
import jax
import jax.numpy as jnp
from jax import lax
from jax.experimental import pallas as pl

N = 400
B = 4
D = 128
EPS = 1e-5
_F32 = jnp.float32


def _ln(x, g, b):
    mu = jnp.mean(x, axis=-1, keepdims=True)
    var = jnp.mean((x - mu) ** 2, axis=-1, keepdims=True)
    return (x - mu) * lax.rsqrt(var + EPS) * g + b


def _dot(a, c):
    return jnp.dot(a, c, preferred_element_type=_F32)


def _tdot(a, c):
    return lax.dot_general(a, c, (((0,), (0,)), ((), ())),
                           preferred_element_type=_F32)


def _fused_kernel(sc_ref, W1_ref, b1_ref, W2_ref, b2_ref, lnEg_ref, lnEb_ref,
                  fc1W_ref, fc1b_ref, ln1g_ref, ln1b_ref,
                  fc2W_ref, fc2b_ref, ln2g_ref, ln2b_ref,
                  fc3W_ref, fc3b_ref,
                  logits_ref, zp_ref):
    b = pl.program_id(0)
    S = sc_ref[0]

    deg = _tdot(S, jnp.ones((N, 1), _F32))
    dinv = jnp.where(deg > 0, lax.rsqrt(deg), 0.0)

    h = _dot(S, W1_ref[...])
    x = jnp.maximum(_tdot(S, h * dinv) * dinv + b1_ref[...], 0.0)
    h = _dot(x, W2_ref[...])
    x = _tdot(S, h * dinv) * dinv + b2_ref[...]
    x = _ln(x, lnEg_ref[...], lnEb_ref[...])
    zp_ref[pl.ds(b, 1), :] = jnp.mean(x, axis=0, keepdims=True)

    @pl.when(b == B - 1)
    def _head():
        z = zp_ref[...]
        hh = _dot(z, fc1W_ref[...]) + fc1b_ref[...]
        hh = jnp.maximum(_ln(hh, ln1g_ref[...], ln1b_ref[...]), 0.0)
        hh = _dot(hh, fc2W_ref[...]) + fc2b_ref[...]
        hh = jnp.maximum(_ln(hh, ln2g_ref[...], ln2b_ref[...]), 0.0)
        logits_ref[...] = _dot(hh, fc3W_ref[...]) + fc3b_ref[...]


def _full(shape):
    return pl.BlockSpec(shape, lambda b: (0,) * len(shape))


def kernel(sc_matrix, W1, b1, W2, b2, lnE_g, lnE_b, fc1_W, fc1_b, ln1_g,
           ln1_b, fc2_W, fc2_b, ln2_g, ln2_b, fc3_W, fc3_b):
    r2 = lambda v: v.reshape(1, -1)
    logits, zp = pl.pallas_call(
        _fused_kernel,
        grid=(B,),
        in_specs=[
            pl.BlockSpec((1, N, N), lambda b: (b, 0, 0)),
            _full((N, D)), _full((1, D)), _full((D, D)), _full((1, D)),
            _full((1, D)), _full((1, D)),
            _full((D, 128)), _full((1, 128)), _full((1, 128)), _full((1, 128)),
            _full((128, 64)), _full((1, 64)), _full((1, 64)), _full((1, 64)),
            _full((64, 4)), _full((1, 4)),
        ],
        out_specs=[
            pl.BlockSpec((B, 4), lambda b: (0, 0)),
            pl.BlockSpec((B, D), lambda b: (0, 0)),
        ],
        out_shape=[
            jax.ShapeDtypeStruct((B, 4), _F32),
            jax.ShapeDtypeStruct((B, D), _F32),
        ],
    )(sc_matrix, W1, r2(b1), W2, r2(b2), r2(lnE_g), r2(lnE_b),
      fc1_W, r2(fc1_b), r2(ln1_g), r2(ln1_b),
      fc2_W, r2(fc2_b), r2(ln2_g), r2(ln2_b),
      fc3_W, r2(fc3_b))
    return (logits, zp)


# speedup vs baseline: 2470.8185x; 1.2316x over previous
import jax
import jax.numpy as jnp
from jax import lax
from jax.experimental import pallas as pl

N = 400
B = 4
D = 128
EPS = 1e-5
_F32 = jnp.float32


def _ln(x, g, b):
    mu = jnp.mean(x, axis=-1, keepdims=True)
    var = jnp.mean((x - mu) ** 2, axis=-1, keepdims=True)
    return (x - mu) * lax.rsqrt(var + EPS) * g + b


def _dot(a, c):
    return jnp.dot(a, c, preferred_element_type=_F32)


def _tdot(a, c):
    return lax.dot_general(a, c, (((0,), (0,)), ((), ())),
                           preferred_element_type=_F32)


def _fused_kernel(sc_ref, W1_ref, b1_ref, W2_ref, b2_ref, lnEg_ref, lnEb_ref,
                  fc1W_ref, fc1b_ref, ln1g_ref, ln1b_ref,
                  fc2W_ref, fc2b_ref, ln2g_ref, ln2b_ref,
                  fc3W_ref, fc3b_ref,
                  logits_ref, zp_ref):
    SS = sc_ref[...]
    Sb = [SS[i * N:(i + 1) * N, :] for i in range(B)]

    ones = jnp.ones((N, 1), _F32)
    dinv = []
    for i in range(B):
        deg = _tdot(Sb[i], ones)
        dinv.append(jnp.where(deg > 0, lax.rsqrt(deg), 0.0))

    h_all = _dot(SS, W1_ref[...])
    x1 = []
    for i in range(B):
        h = h_all[i * N:(i + 1) * N, :]
        x1.append(jnp.maximum(
            _tdot(Sb[i], h * dinv[i]) * dinv[i] + b1_ref[...], 0.0))

    h2_all = _dot(jnp.concatenate(x1, axis=0), W2_ref[...])
    for i in range(B):
        h = h2_all[i * N:(i + 1) * N, :]
        y = _tdot(Sb[i], h * dinv[i]) * dinv[i] + b2_ref[...]
        y = _ln(y, lnEg_ref[...], lnEb_ref[...])
        zp_ref[pl.ds(i, 1), :] = jnp.mean(y, axis=0, keepdims=True)

    z = zp_ref[...]
    hh = _dot(z, fc1W_ref[...]) + fc1b_ref[...]
    hh = jnp.maximum(_ln(hh, ln1g_ref[...], ln1b_ref[...]), 0.0)
    hh = _dot(hh, fc2W_ref[...]) + fc2b_ref[...]
    hh = jnp.maximum(_ln(hh, ln2g_ref[...], ln2b_ref[...]), 0.0)
    logits_ref[...] = _dot(hh, fc3W_ref[...]) + fc3b_ref[...]


def kernel(sc_matrix, W1, b1, W2, b2, lnE_g, lnE_b, fc1_W, fc1_b, ln1_g,
           ln1_b, fc2_W, fc2_b, ln2_g, ln2_b, fc3_W, fc3_b):
    r2 = lambda v: v.reshape(1, -1)
    logits, zp = pl.pallas_call(
        _fused_kernel,
        out_shape=[
            jax.ShapeDtypeStruct((B, 4), _F32),
            jax.ShapeDtypeStruct((B, D), _F32),
        ],
    )(sc_matrix.reshape(B * N, N), W1, r2(b1), W2, r2(b2), r2(lnE_g),
      r2(lnE_b), fc1_W, r2(fc1_b), r2(ln1_g), r2(ln1_b),
      fc2_W, r2(fc2_b), r2(ln2_g), r2(ln2_b),
      fc3_W, r2(fc3_b))
    return (logits, zp)
